# baseline (speedup 1.0000x reference)
import functools

import jax
import jax.numpy as jnp
from jax import lax
from jax.experimental import pallas as pl
from jax.experimental.pallas import tpu as pltpu
from jax.experimental.pallas import tpu_sc as plsc

_N = 10000
_D = 128
_E = 320000
_CHUNK = 128
_CORES = 2
_SUBCORES = 16
_TILES = _CORES * _SUBCORES
_NCH = 79
_EPT = _NCH * _CHUNK
_E_PAD = _TILES * _EPT
_ROW_GROUPS = _N // 8
_LANES = 16
_NROW = 2
_NIDX = 4


def _sc_spmm_partials(x, row_p, col_p, a_p):
    mesh = plsc.VectorSubcoreMesh(core_axis_name="c", subcore_axis_name="s")

    @functools.partial(
        pl.kernel,
        mesh=mesh,
        out_type=jax.ShapeDtypeStruct((_CORES, _N, _D), jnp.float32),
        scratch_types=(
            [pltpu.VMEM((_CHUNK, _D), jnp.float32)] * _NROW
            + [pltpu.VMEM((_CHUNK,), jnp.int32)] * _NIDX
            + [pltpu.VMEM((_CHUNK,), jnp.int32)] * _NIDX
            + [pltpu.VMEM((_CHUNK,), jnp.float32)] * _NIDX
            + [pltpu.VMEM_SHARED((_N, _D), jnp.float32)]
            + [pltpu.SemaphoreType.DMA] * (_NROW + _NROW + _NIDX)
        ),
    )
    def k(x_hbm, row_hbm, col_hbm, a_hbm, out_hbm, *refs):
        rows = refs[0:_NROW]
        csl = refs[_NROW:_NROW + _NIDX]
        dsl = refs[_NROW + _NIDX:_NROW + 2 * _NIDX]
        asl = refs[_NROW + 2 * _NIDX:_NROW + 3 * _NIDX]
        f_sh = refs[_NROW + 3 * _NIDX]
        gsem = refs[_NROW + 3 * _NIDX + 1:_NROW + 3 * _NIDX + 1 + _NROW]
        ssem = refs[_NROW + 3 * _NIDX + 1 + _NROW:
                    _NROW + 3 * _NIDX + 1 + 2 * _NROW]
        isem = refs[_NROW + 3 * _NIDX + 1 + 2 * _NROW:]
        cid = lax.axis_index("c")
        sid = lax.axis_index("s")
        w = cid * _SUBCORES + sid
        eb0 = w * _EPT

        for r in range(8):
            for cc in range(_D // _LANES):
                rows[0][r, pl.ds(cc * _LANES, _LANES)] = jnp.zeros(
                    (_LANES,), jnp.float32)
        ngrp = jnp.where(sid < _ROW_GROUPS % _SUBCORES,
                         _ROW_GROUPS // _SUBCORES + 1,
                         _ROW_GROUPS // _SUBCORES)
        gbase = sid * (_ROW_GROUPS // _SUBCORES) + jnp.minimum(
            sid, _ROW_GROUPS % _SUBCORES)

        def zero_grp(g, carry):
            pltpu.sync_copy(
                rows[0].at[pl.ds(0, 8), :],
                f_sh.at[pl.ds((gbase + g) * 8, 8), :])
            return carry

        lax.fori_loop(0, ngrp, zero_grp, 0)
        plsc.subcore_barrier()

        def idx_copies(j, si):
            eb = eb0 + j * _CHUNK
            return (
                pltpu.make_async_copy(
                    col_hbm.at[pl.ds(eb, _CHUNK)], csl[si], isem[si]),
                pltpu.make_async_copy(
                    row_hbm.at[pl.ds(eb, _CHUNK)], dsl[si], isem[si]),
                pltpu.make_async_copy(
                    a_hbm.at[pl.ds(eb, _CHUNK)], asl[si], isem[si]),
            )

        def gather(si, sr):
            return pltpu.make_async_copy(
                x_hbm.at[csl[si]], rows[sr], gsem[sr])

        def scatter(si, sr):
            return pltpu.make_async_copy(
                rows[sr], f_sh.at[dsl[si]], ssem[sr])

        for c in idx_copies(0, 0):
            c.start()
        for c in idx_copies(1, 1):
            c.start()
        for c in idx_copies(0, 0):
            c.wait()
        gather(0, 0).start()

        def chunk_body(i, carry):
            s4 = lax.rem(i, _NIDX)

            for s in range(_NIDX):
                sr = s % _NROW
                srn = (s + 1) % _NROW
                sin = (s + 1) % _NIDX
                si2 = (s + 2) % _NIDX

                @pl.when(s4 == s)
                def _(s=s, sr=sr, srn=srn, sin=sin, si2=si2):
                    @pl.when(i + 1 < _NCH)
                    def _():
                        for c in idx_copies(i + 1, sin):
                            c.wait()

                        @pl.when(i >= 1)
                        def _():
                            scatter(sin, srn).wait()

                        gather(sin, srn).start()

                    @pl.when(i + 2 < _NCH)
                    def _():
                        for c in idx_copies(i + 2, si2):
                            c.start()

                    gather(s, sr).wait()

                    @plsc.parallel_loop(0, _CHUNK // _LANES, unroll=2)
                    def _(g, s=s, sr=sr):
                        a16 = asl[s][pl.ds(g * _LANES, _LANES)]
                        for j in range(_LANES):
                            ab = jnp.broadcast_to(a16[j], (_LANES,))
                            r = g * _LANES + j
                            for cc in range(_D // _LANES):
                                sl = pl.ds(cc * _LANES, _LANES)
                                rows[sr][r, sl] = rows[sr][r, sl] * ab

                    pltpu.async_copy(
                        rows[sr], f_sh.at[dsl[s]], ssem[sr], add=True)

            return carry

        lax.fori_loop(0, _NCH, chunk_body, 0)

        scatter((_NCH - 1) % _NIDX, (_NCH - 1) % _NROW).wait()
        scatter((_NCH - 2) % _NIDX, (_NCH - 2) % _NROW).wait()

        plsc.subcore_barrier()

        def write_grp(g, carry):
            rb = (gbase + g) * 8
            pltpu.sync_copy(
                f_sh.at[pl.ds(rb, 8), :],
                out_hbm.at[cid, pl.ds(rb, 8), :])
            return carry

        lax.fori_loop(0, ngrp, write_grp, 0)

    return k(x, row_p, col_p, a_p)


def _combine_relu(partials):
    blk = 1000

    def body(p_ref, o_ref):
        o_ref[...] = jnp.maximum(p_ref[0] + p_ref[1], 0.0)

    return pl.pallas_call(
        body,
        grid=(_N // blk,),
        in_specs=[pl.BlockSpec((_CORES, blk, _D), lambda i: (0, i, 0))],
        out_specs=pl.BlockSpec((blk, _D), lambda i: (i, 0)),
        out_shape=jax.ShapeDtypeStruct((_N, _D), jnp.float32),
    )(partials)


def kernel(t, x, edge_index, A_vals):
    npad = _E_PAD - _E
    row_p = jnp.concatenate(
        [edge_index[0], jnp.zeros((npad,), jnp.int32)])
    col_p = jnp.concatenate(
        [edge_index[1], jnp.zeros((npad,), jnp.int32)])
    a_p = jnp.concatenate([A_vals, jnp.zeros((npad,), jnp.float32)])
    partials = _sc_spmm_partials(x, row_p, col_p, a_p)
    return _combine_relu(partials)


# speedup vs baseline: 1.0971x; 1.0971x over previous
import functools

import jax
import jax.numpy as jnp
from jax import lax
from jax.experimental import pallas as pl
from jax.experimental.pallas import tpu as pltpu
from jax.experimental.pallas import tpu_sc as plsc

_N = 10000
_D = 128
_E = 320000
_CHUNK = 128
_CORES = 2
_SUBCORES = 16
_TILES = _CORES * _SUBCORES
_NCH0 = 104
_NCH1 = 54
_E_PAD = _SUBCORES * (_NCH0 + _NCH1) * _CHUNK
_ROW_GROUPS = _N // 8
_LANES = 16
_NROW = 2
_NIDX = 4


def _sc_spmm_partials(x, row_p, col_p, a_p):
    mesh = plsc.VectorSubcoreMesh(core_axis_name="c", subcore_axis_name="s")

    @functools.partial(
        pl.kernel,
        mesh=mesh,
        out_type=jax.ShapeDtypeStruct((_CORES, _N, _D), jnp.float32),
        scratch_types=(
            [pltpu.VMEM((_CHUNK, _D), jnp.float32)] * _NROW
            + [pltpu.VMEM((_CHUNK,), jnp.int32)] * _NIDX
            + [pltpu.VMEM((_CHUNK,), jnp.int32)] * _NIDX
            + [pltpu.VMEM((_CHUNK,), jnp.float32)] * _NIDX
            + [pltpu.VMEM_SHARED((_N, _D), jnp.float32)]
            + [pltpu.SemaphoreType.DMA] * (_NROW + _NROW + _NIDX)
        ),
    )
    def k(x_hbm, row_hbm, col_hbm, a_hbm, out_hbm, *refs):
        rows = refs[0:_NROW]
        csl = refs[_NROW:_NROW + _NIDX]
        dsl = refs[_NROW + _NIDX:_NROW + 2 * _NIDX]
        asl = refs[_NROW + 2 * _NIDX:_NROW + 3 * _NIDX]
        f_sh = refs[_NROW + 3 * _NIDX]
        gsem = refs[_NROW + 3 * _NIDX + 1:_NROW + 3 * _NIDX + 1 + _NROW]
        ssem = refs[_NROW + 3 * _NIDX + 1 + _NROW:
                    _NROW + 3 * _NIDX + 1 + 2 * _NROW]
        isem = refs[_NROW + 3 * _NIDX + 1 + 2 * _NROW:]
        cid = lax.axis_index("c")
        sid = lax.axis_index("s")
        nch = jnp.where(cid == 0, _NCH0, _NCH1)
        base_chunk = jnp.where(cid == 0, sid * _NCH0,
                               _SUBCORES * _NCH0 + sid * _NCH1)
        eb0 = base_chunk * _CHUNK

        for r in range(8):
            for cc in range(_D // _LANES):
                rows[0][r, pl.ds(cc * _LANES, _LANES)] = jnp.zeros(
                    (_LANES,), jnp.float32)
        ngrp = jnp.where(sid < _ROW_GROUPS % _SUBCORES,
                         _ROW_GROUPS // _SUBCORES + 1,
                         _ROW_GROUPS // _SUBCORES)
        gbase = sid * (_ROW_GROUPS // _SUBCORES) + jnp.minimum(
            sid, _ROW_GROUPS % _SUBCORES)

        def zero_grp(g, carry):
            pltpu.sync_copy(
                rows[0].at[pl.ds(0, 8), :],
                f_sh.at[pl.ds((gbase + g) * 8, 8), :])
            return carry

        lax.fori_loop(0, ngrp, zero_grp, 0)
        plsc.subcore_barrier()

        def idx_copies(j, si):
            eb = eb0 + j * _CHUNK
            return (
                pltpu.make_async_copy(
                    col_hbm.at[pl.ds(eb, _CHUNK)], csl[si], isem[si]),
                pltpu.make_async_copy(
                    row_hbm.at[pl.ds(eb, _CHUNK)], dsl[si], isem[si]),
                pltpu.make_async_copy(
                    a_hbm.at[pl.ds(eb, _CHUNK)], asl[si], isem[si]),
            )

        def gather(si, sr):
            return pltpu.make_async_copy(
                x_hbm.at[csl[si]], rows[sr], gsem[sr])

        def scatter(si, sr):
            return pltpu.make_async_copy(
                rows[sr], f_sh.at[dsl[si]], ssem[sr])

        for c in idx_copies(0, 0):
            c.start()
        for c in idx_copies(1, 1):
            c.start()
        for c in idx_copies(0, 0):
            c.wait()
        gather(0, 0).start()

        def chunk_body(i, carry):
            s4 = lax.rem(i, _NIDX)

            for s in range(_NIDX):
                sr = s % _NROW
                srn = (s + 1) % _NROW
                sin = (s + 1) % _NIDX
                si2 = (s + 2) % _NIDX

                @pl.when(s4 == s)
                def _(s=s, sr=sr, srn=srn, sin=sin, si2=si2):
                    @pl.when(i + 1 < nch)
                    def _():
                        for c in idx_copies(i + 1, sin):
                            c.wait()

                        @pl.when(i >= 1)
                        def _():
                            scatter(sin, srn).wait()

                        gather(sin, srn).start()

                    @pl.when(i + 2 < nch)
                    def _():
                        for c in idx_copies(i + 2, si2):
                            c.start()

                    gather(s, sr).wait()

                    @plsc.parallel_loop(0, _CHUNK // _LANES, unroll=2)
                    def _(g, s=s, sr=sr):
                        a16 = asl[s][pl.ds(g * _LANES, _LANES)]
                        for j in range(_LANES):
                            ab = jnp.broadcast_to(a16[j], (_LANES,))
                            r = g * _LANES + j
                            for cc in range(_D // _LANES):
                                sl = pl.ds(cc * _LANES, _LANES)
                                rows[sr][r, sl] = rows[sr][r, sl] * ab

                    pltpu.async_copy(
                        rows[sr], f_sh.at[dsl[s]], ssem[sr], add=True)

            return carry

        lax.fori_loop(0, nch, chunk_body, 0)

        scatter(0, 0).wait()
        scatter(1, 1).wait()

        plsc.subcore_barrier()

        def write_grp(g, carry):
            rb = (gbase + g) * 8
            pltpu.sync_copy(
                f_sh.at[pl.ds(rb, 8), :],
                out_hbm.at[cid, pl.ds(rb, 8), :])
            return carry

        lax.fori_loop(0, ngrp, write_grp, 0)

    return k(x, row_p, col_p, a_p)


def _combine_relu(partials):
    blk = 1000

    def body(p_ref, o_ref):
        o_ref[...] = jnp.maximum(p_ref[0] + p_ref[1], 0.0)

    return pl.pallas_call(
        body,
        grid=(_N // blk,),
        in_specs=[pl.BlockSpec((_CORES, blk, _D), lambda i: (0, i, 0))],
        out_specs=pl.BlockSpec((blk, _D), lambda i: (i, 0)),
        out_shape=jax.ShapeDtypeStruct((_N, _D), jnp.float32),
    )(partials)


def kernel(t, x, edge_index, A_vals):
    npad = _E_PAD - _E
    row_p = jnp.concatenate(
        [edge_index[0], jnp.zeros((npad,), jnp.int32)])
    col_p = jnp.concatenate(
        [edge_index[1], jnp.zeros((npad,), jnp.int32)])
    a_p = jnp.concatenate([A_vals, jnp.zeros((npad,), jnp.float32)])
    partials = _sc_spmm_partials(x, row_p, col_p, a_p)
    return _combine_relu(partials)


# speedup vs baseline: 1.2352x; 1.1259x over previous
import functools

import jax
import jax.numpy as jnp
from jax import lax
from jax.experimental import pallas as pl
from jax.experimental.pallas import tpu as pltpu
from jax.experimental.pallas import tpu_sc as plsc

_N = 10000
_D = 128
_E = 320000
_CHUNK = 128
_CORES = 2
_SUBCORES = 16
_TILES = _CORES * _SUBCORES
_NCH0 = 136
_NCH1 = 22
_E_PAD = _SUBCORES * (_NCH0 + _NCH1) * _CHUNK
_ROW_GROUPS = _N // 8
_LANES = 16
_NROW = 2
_NIDX = 4


def _sc_spmm_partials(x, row_p, col_p, a_p):
    mesh = plsc.VectorSubcoreMesh(core_axis_name="c", subcore_axis_name="s")

    @functools.partial(
        pl.kernel,
        mesh=mesh,
        out_type=jax.ShapeDtypeStruct((_CORES, _N, _D), jnp.float32),
        scratch_types=(
            [pltpu.VMEM((_CHUNK, _D), jnp.float32)] * _NROW
            + [pltpu.VMEM((_CHUNK,), jnp.int32)] * _NIDX
            + [pltpu.VMEM((_CHUNK,), jnp.int32)] * _NIDX
            + [pltpu.VMEM((_CHUNK,), jnp.float32)] * _NIDX
            + [pltpu.VMEM_SHARED((_N, _D), jnp.float32)]
            + [pltpu.SemaphoreType.DMA] * (_NROW + _NROW + _NIDX)
        ),
    )
    def k(x_hbm, row_hbm, col_hbm, a_hbm, out_hbm, *refs):
        rows = refs[0:_NROW]
        csl = refs[_NROW:_NROW + _NIDX]
        dsl = refs[_NROW + _NIDX:_NROW + 2 * _NIDX]
        asl = refs[_NROW + 2 * _NIDX:_NROW + 3 * _NIDX]
        f_sh = refs[_NROW + 3 * _NIDX]
        gsem = refs[_NROW + 3 * _NIDX + 1:_NROW + 3 * _NIDX + 1 + _NROW]
        ssem = refs[_NROW + 3 * _NIDX + 1 + _NROW:
                    _NROW + 3 * _NIDX + 1 + 2 * _NROW]
        isem = refs[_NROW + 3 * _NIDX + 1 + 2 * _NROW:]
        cid = lax.axis_index("c")
        sid = lax.axis_index("s")
        nch = jnp.where(cid == 0, _NCH0, _NCH1)
        base_chunk = jnp.where(cid == 0, sid * _NCH0,
                               _SUBCORES * _NCH0 + sid * _NCH1)
        eb0 = base_chunk * _CHUNK

        for r in range(8):
            for cc in range(_D // _LANES):
                rows[0][r, pl.ds(cc * _LANES, _LANES)] = jnp.zeros(
                    (_LANES,), jnp.float32)
        ngrp = jnp.where(sid < _ROW_GROUPS % _SUBCORES,
                         _ROW_GROUPS // _SUBCORES + 1,
                         _ROW_GROUPS // _SUBCORES)
        gbase = sid * (_ROW_GROUPS // _SUBCORES) + jnp.minimum(
            sid, _ROW_GROUPS % _SUBCORES)

        def zero_grp(g, carry):
            pltpu.sync_copy(
                rows[0].at[pl.ds(0, 8), :],
                f_sh.at[pl.ds((gbase + g) * 8, 8), :])
            return carry

        lax.fori_loop(0, ngrp, zero_grp, 0)
        plsc.subcore_barrier()

        def idx_copies(j, si):
            eb = eb0 + j * _CHUNK
            return (
                pltpu.make_async_copy(
                    col_hbm.at[pl.ds(eb, _CHUNK)], csl[si], isem[si]),
                pltpu.make_async_copy(
                    row_hbm.at[pl.ds(eb, _CHUNK)], dsl[si], isem[si]),
                pltpu.make_async_copy(
                    a_hbm.at[pl.ds(eb, _CHUNK)], asl[si], isem[si]),
            )

        def gather(si, sr):
            return pltpu.make_async_copy(
                x_hbm.at[csl[si]], rows[sr], gsem[sr])

        def scatter(si, sr):
            return pltpu.make_async_copy(
                rows[sr], f_sh.at[dsl[si]], ssem[sr])

        for c in idx_copies(0, 0):
            c.start()
        for c in idx_copies(1, 1):
            c.start()
        for c in idx_copies(0, 0):
            c.wait()
        gather(0, 0).start()

        def chunk_body(i, carry):
            s4 = lax.rem(i, _NIDX)

            for s in range(_NIDX):
                sr = s % _NROW
                srn = (s + 1) % _NROW
                sin = (s + 1) % _NIDX
                si2 = (s + 2) % _NIDX

                @pl.when(s4 == s)
                def _(s=s, sr=sr, srn=srn, sin=sin, si2=si2):
                    @pl.when(i + 1 < nch)
                    def _():
                        for c in idx_copies(i + 1, sin):
                            c.wait()

                        @pl.when(i >= 1)
                        def _():
                            scatter(sin, srn).wait()

                        gather(sin, srn).start()

                    @pl.when(i + 2 < nch)
                    def _():
                        for c in idx_copies(i + 2, si2):
                            c.start()

                    gather(s, sr).wait()

                    @plsc.parallel_loop(0, _CHUNK // _LANES, unroll=2)
                    def _(g, s=s, sr=sr):
                        a16 = asl[s][pl.ds(g * _LANES, _LANES)]
                        for j in range(_LANES):
                            ab = jnp.broadcast_to(a16[j], (_LANES,))
                            r = g * _LANES + j
                            for cc in range(_D // _LANES):
                                sl = pl.ds(cc * _LANES, _LANES)
                                rows[sr][r, sl] = rows[sr][r, sl] * ab

                    pltpu.async_copy(
                        rows[sr], f_sh.at[dsl[s]], ssem[sr], add=True)

            return carry

        lax.fori_loop(0, nch, chunk_body, 0)

        scatter(0, 0).wait()
        scatter(1, 1).wait()

        plsc.subcore_barrier()

        def write_grp(g, carry):
            rb = (gbase + g) * 8
            pltpu.sync_copy(
                f_sh.at[pl.ds(rb, 8), :],
                out_hbm.at[cid, pl.ds(rb, 8), :])
            return carry

        lax.fori_loop(0, ngrp, write_grp, 0)

    return k(x, row_p, col_p, a_p)


def _combine_relu(partials):
    blk = 1000

    def body(p_ref, o_ref):
        o_ref[...] = jnp.maximum(p_ref[0] + p_ref[1], 0.0)

    return pl.pallas_call(
        body,
        grid=(_N // blk,),
        in_specs=[pl.BlockSpec((_CORES, blk, _D), lambda i: (0, i, 0))],
        out_specs=pl.BlockSpec((blk, _D), lambda i: (i, 0)),
        out_shape=jax.ShapeDtypeStruct((_N, _D), jnp.float32),
    )(partials)


def kernel(t, x, edge_index, A_vals):
    npad = _E_PAD - _E
    row_p = jnp.concatenate(
        [edge_index[0], jnp.zeros((npad,), jnp.int32)])
    col_p = jnp.concatenate(
        [edge_index[1], jnp.zeros((npad,), jnp.int32)])
    a_p = jnp.concatenate([A_vals, jnp.zeros((npad,), jnp.float32)])
    partials = _sc_spmm_partials(x, row_p, col_p, a_p)
    return _combine_relu(partials)


# speedup vs baseline: 1.2387x; 1.0028x over previous
import functools

import jax
import jax.numpy as jnp
from jax import lax
from jax.experimental import pallas as pl
from jax.experimental.pallas import tpu as pltpu
from jax.experimental.pallas import tpu_sc as plsc

_N = 10000
_D = 128
_E = 320000
_CHUNK = 128
_CORES = 2
_SUBCORES = 16
_TILES = _CORES * _SUBCORES
_NCH0 = 138
_NCH1 = 20
_E_PAD = _SUBCORES * (_NCH0 + _NCH1) * _CHUNK
_ROW_GROUPS = _N // 8
_LANES = 16
_NROW = 2
_NIDX = 4


def _sc_spmm_partials(x, row_p, col_p, a_p):
    mesh = plsc.VectorSubcoreMesh(core_axis_name="c", subcore_axis_name="s")

    @functools.partial(
        pl.kernel,
        mesh=mesh,
        out_type=jax.ShapeDtypeStruct((_CORES, _N, _D), jnp.float32),
        scratch_types=(
            [pltpu.VMEM((_CHUNK, _D), jnp.float32)] * _NROW
            + [pltpu.VMEM((_CHUNK,), jnp.int32)] * _NIDX
            + [pltpu.VMEM((_CHUNK,), jnp.int32)] * _NIDX
            + [pltpu.VMEM((_CHUNK,), jnp.float32)] * _NIDX
            + [pltpu.VMEM_SHARED((_N, _D), jnp.float32)]
            + [pltpu.SemaphoreType.DMA] * (_NROW + _NROW + _NIDX)
        ),
    )
    def k(x_hbm, row_hbm, col_hbm, a_hbm, out_hbm, *refs):
        rows = refs[0:_NROW]
        csl = refs[_NROW:_NROW + _NIDX]
        dsl = refs[_NROW + _NIDX:_NROW + 2 * _NIDX]
        asl = refs[_NROW + 2 * _NIDX:_NROW + 3 * _NIDX]
        f_sh = refs[_NROW + 3 * _NIDX]
        gsem = refs[_NROW + 3 * _NIDX + 1:_NROW + 3 * _NIDX + 1 + _NROW]
        ssem = refs[_NROW + 3 * _NIDX + 1 + _NROW:
                    _NROW + 3 * _NIDX + 1 + 2 * _NROW]
        isem = refs[_NROW + 3 * _NIDX + 1 + 2 * _NROW:]
        cid = lax.axis_index("c")
        sid = lax.axis_index("s")
        nch = jnp.where(cid == 0, _NCH0, _NCH1)
        base_chunk = jnp.where(cid == 0, sid * _NCH0,
                               _SUBCORES * _NCH0 + sid * _NCH1)
        eb0 = base_chunk * _CHUNK

        for r in range(8):
            for cc in range(_D // _LANES):
                rows[0][r, pl.ds(cc * _LANES, _LANES)] = jnp.zeros(
                    (_LANES,), jnp.float32)
        ngrp = jnp.where(sid < _ROW_GROUPS % _SUBCORES,
                         _ROW_GROUPS // _SUBCORES + 1,
                         _ROW_GROUPS // _SUBCORES)
        gbase = sid * (_ROW_GROUPS // _SUBCORES) + jnp.minimum(
            sid, _ROW_GROUPS % _SUBCORES)

        def zero_grp(g, carry):
            pltpu.sync_copy(
                rows[0].at[pl.ds(0, 8), :],
                f_sh.at[pl.ds((gbase + g) * 8, 8), :])
            return carry

        lax.fori_loop(0, ngrp, zero_grp, 0)
        plsc.subcore_barrier()

        def idx_copies(j, si):
            eb = eb0 + j * _CHUNK
            return (
                pltpu.make_async_copy(
                    col_hbm.at[pl.ds(eb, _CHUNK)], csl[si], isem[si]),
                pltpu.make_async_copy(
                    row_hbm.at[pl.ds(eb, _CHUNK)], dsl[si], isem[si]),
                pltpu.make_async_copy(
                    a_hbm.at[pl.ds(eb, _CHUNK)], asl[si], isem[si]),
            )

        def gather(si, sr):
            return pltpu.make_async_copy(
                x_hbm.at[csl[si]], rows[sr], gsem[sr])

        def scatter(si, sr):
            return pltpu.make_async_copy(
                rows[sr], f_sh.at[dsl[si]], ssem[sr])

        for c in idx_copies(0, 0):
            c.start()
        for c in idx_copies(1, 1):
            c.start()
        for c in idx_copies(0, 0):
            c.wait()
        gather(0, 0).start()

        def chunk_body(i, carry):
            s4 = lax.rem(i, _NIDX)

            for s in range(_NIDX):
                sr = s % _NROW
                srn = (s + 1) % _NROW
                sin = (s + 1) % _NIDX
                si2 = (s + 2) % _NIDX

                @pl.when(s4 == s)
                def _(s=s, sr=sr, srn=srn, sin=sin, si2=si2):
                    @pl.when(i + 1 < nch)
                    def _():
                        for c in idx_copies(i + 1, sin):
                            c.wait()

                        @pl.when(i >= 1)
                        def _():
                            scatter(sin, srn).wait()

                        gather(sin, srn).start()

                    @pl.when(i + 2 < nch)
                    def _():
                        for c in idx_copies(i + 2, si2):
                            c.start()

                    gather(s, sr).wait()

                    @plsc.parallel_loop(0, _CHUNK // _LANES, unroll=2)
                    def _(g, s=s, sr=sr):
                        a16 = asl[s][pl.ds(g * _LANES, _LANES)]
                        for j in range(_LANES):
                            ab = jnp.broadcast_to(a16[j], (_LANES,))
                            r = g * _LANES + j
                            for cc in range(_D // _LANES):
                                sl = pl.ds(cc * _LANES, _LANES)
                                rows[sr][r, sl] = rows[sr][r, sl] * ab

                    pltpu.async_copy(
                        rows[sr], f_sh.at[dsl[s]], ssem[sr], add=True)

            return carry

        lax.fori_loop(0, nch, chunk_body, 0)

        scatter(0, 0).wait()
        scatter(1, 1).wait()

        plsc.subcore_barrier()

        def write_grp(g, carry):
            rb = (gbase + g) * 8
            pltpu.sync_copy(
                f_sh.at[pl.ds(rb, 8), :],
                out_hbm.at[cid, pl.ds(rb, 8), :])
            return carry

        lax.fori_loop(0, ngrp, write_grp, 0)

    return k(x, row_p, col_p, a_p)


def _combine_relu(partials):
    blk = 1000

    def body(p_ref, o_ref):
        o_ref[...] = jnp.maximum(p_ref[0] + p_ref[1], 0.0)

    return pl.pallas_call(
        body,
        grid=(_N // blk,),
        in_specs=[pl.BlockSpec((_CORES, blk, _D), lambda i: (0, i, 0))],
        out_specs=pl.BlockSpec((blk, _D), lambda i: (i, 0)),
        out_shape=jax.ShapeDtypeStruct((_N, _D), jnp.float32),
    )(partials)


def kernel(t, x, edge_index, A_vals):
    npad = _E_PAD - _E
    row_p = jnp.concatenate(
        [edge_index[0], jnp.zeros((npad,), jnp.int32)])
    col_p = jnp.concatenate(
        [edge_index[1], jnp.zeros((npad,), jnp.int32)])
    a_p = jnp.concatenate([A_vals, jnp.zeros((npad,), jnp.float32)])
    partials = _sc_spmm_partials(x, row_p, col_p, a_p)
    return _combine_relu(partials)
